# Initial kernel scaffold; baseline (speedup 1.0000x reference)
import jax
import jax.numpy as jnp
from jax.experimental import pallas as pl


def kernel(hidden_states, gate_weight, gate_up_weight, down_weight):
    raise NotImplementedError("write your pallas kernel here")


# speedup vs baseline: 3.1078x; 3.1078x over previous
import functools

import jax
import jax.numpy as jnp
from jax.experimental import pallas as pl
from jax.experimental.pallas import tpu as pltpu

E = 64
K = 2
H = 1024
F = 512
T = 2048


def _moe_body(x_ref, gw_ref, guw_ref, dw_ref, out_ref, w_scr):
    e = pl.program_id(0)

    @pl.when(e == 0)
    def _router():
        logits = jnp.dot(x_ref[...], gw_ref[...],
                         preferred_element_type=jnp.float32)
        p = jax.nn.softmax(logits, axis=-1)
        m1 = jnp.max(p, axis=-1, keepdims=True)
        is1 = (p >= m1).astype(p.dtype)
        p2 = p - is1 * 2.0
        m2 = jnp.max(p2, axis=-1, keepdims=True)
        is2 = (p2 >= m2).astype(p.dtype)
        mask = is1 + is2
        s = m1 + m2
        w_scr[...] = p * mask / s

    x = x_ref[...]
    gu = jnp.dot(x, guw_ref[0], preferred_element_type=jnp.float32)
    g = gu[:, :F]
    u = gu[:, F:]
    h = g * jax.nn.sigmoid(g) * u
    o = jnp.dot(h, dw_ref[0], preferred_element_type=jnp.float32)
    lane = jax.lax.broadcasted_iota(jnp.int32, (T, E), 1)
    w = jnp.sum(jnp.where(lane == e, w_scr[...], 0.0), axis=1, keepdims=True)
    contrib = o * w

    @pl.when(e == 0)
    def _init():
        out_ref[...] = contrib

    @pl.when(e > 0)
    def _acc():
        out_ref[...] += contrib


def kernel(hidden_states, gate_weight, gate_up_weight, down_weight):
    return pl.pallas_call(
        _moe_body,
        grid=(E,),
        in_specs=[
            pl.BlockSpec((T, H), lambda e: (0, 0)),
            pl.BlockSpec((H, E), lambda e: (0, 0)),
            pl.BlockSpec((1, H, 2 * F), lambda e: (e, 0, 0)),
            pl.BlockSpec((1, F, H), lambda e: (e, 0, 0)),
        ],
        out_specs=pl.BlockSpec((T, H), lambda e: (0, 0)),
        out_shape=jax.ShapeDtypeStruct((T, H), jnp.float32),
        scratch_shapes=[pltpu.VMEM((T, E), jnp.float32)],
    )(hidden_states, gate_weight, gate_up_weight, down_weight)
